# baseline (speedup 1.0000x reference)
import jax
import jax.numpy as jnp
from jax.experimental import pallas as pl
from jax.experimental.pallas import tpu as pltpu

_MARGIN = 0.8
_BLK = 512
_BIG = 1e30


def _triplet_kernel(a_ref, e_ref, key_ref, sbj_ref, sum_ref, cnt_ref):
    i = pl.program_id(0)
    a = a_ref[...]
    e = e_ref[...]
    B = e.shape[0]
    blk = a.shape[0]

    sq_blk = jnp.sum(a * a, axis=1, keepdims=True)
    sq_all = jnp.sum(e * e, axis=1, keepdims=True)
    a_aug = jnp.concatenate(
        [a, sq_blk, jnp.ones((blk, 1), jnp.float32)], axis=1)
    e_aug = jnp.concatenate(
        [-2.0 * e, jnp.ones((B, 1), jnp.float32), sq_all], axis=1)
    d2 = jax.lax.dot_general(
        a_aug, e_aug, (((1,), (1,)), ((), ())),
        preferred_element_type=jnp.float32)

    key = key_ref[0, :]
    sbj = sbj_ref[0, :]
    key_r = key_ref[0, pl.ds(i * blk, blk)]
    sbj_r = sbj_ref[0, pl.ds(i * blk, blk)]

    key_eq = key_r[:, None] == key[None, :]
    sbj_eq = sbj_r[:, None] == sbj[None, :]
    row = i * blk + jax.lax.broadcasted_iota(jnp.int32, (blk, B), 0)
    col = jax.lax.broadcasted_iota(jnp.int32, (blk, B), 1)
    pos = key_eq & (row != col)
    neg = sbj_eq & jnp.logical_not(key_eq)

    dpos2 = jnp.max(jnp.where(pos, d2, -1.0), axis=1)
    dneg2 = jnp.min(jnp.where(neg, d2, _BIG), axis=1)
    valid = (dpos2 >= 0.0) & (dneg2 < 1e29)

    dp = jnp.sqrt(jnp.maximum(dpos2, 0.0))
    dn = jnp.sqrt(jnp.maximum(dneg2, 0.0))
    per = jnp.maximum(dp - dn + _MARGIN, 0.0)
    psum = jnp.sum(jnp.where(valid, per, 0.0))
    pcnt = jnp.sum(valid.astype(jnp.float32))

    @pl.when(i == 0)
    def _():
        sum_ref[...] = jnp.zeros((1, 1), jnp.float32)
        cnt_ref[...] = jnp.zeros((1, 1), jnp.float32)

    sum_ref[...] += psum.reshape(1, 1)
    cnt_ref[...] += pcnt.reshape(1, 1)


def kernel(emb, labels, sbj):
    B, D = emb.shape
    lbl32 = labels.astype(jnp.int32)
    sbj32 = sbj.astype(jnp.int32)
    perm = jnp.argsort(sbj32).astype(jnp.int32)
    emb = jnp.take(emb, perm, axis=0)
    key2 = jnp.take(sbj32 * 8 + lbl32, perm).reshape(1, B)
    sbj2 = jnp.take(sbj32, perm).reshape(1, B)
    grid = B // _BLK
    s, c = pl.pallas_call(
        _triplet_kernel,
        grid=(grid,),
        in_specs=[
            pl.BlockSpec((_BLK, D), lambda i: (i, 0)),
            pl.BlockSpec((B, D), lambda i: (0, 0)),
            pl.BlockSpec((1, B), lambda i: (0, 0)),
            pl.BlockSpec((1, B), lambda i: (0, 0)),
        ],
        out_specs=[
            pl.BlockSpec((1, 1), lambda i: (0, 0)),
            pl.BlockSpec((1, 1), lambda i: (0, 0)),
        ],
        out_shape=[
            jax.ShapeDtypeStruct((1, 1), jnp.float32),
            jax.ShapeDtypeStruct((1, 1), jnp.float32),
        ],
    )(emb, emb, key2, sbj2)
    return s[0, 0] / jnp.maximum(c[0, 0], 1.0)


# speedup vs baseline: 1.5257x; 1.5257x over previous
import numpy as np

import jax
import jax.numpy as jnp
from jax.experimental import pallas as pl
from jax.experimental.pallas import tpu as pltpu

_MARGIN = 0.8
_BLK = 512
_NB = 8
_BIG = 1e30

_PAIRS = [(i, j) for i in range(_NB) for j in range(_NB) if j >= i]
_NPAIR = len(_PAIRS)


def _triplet_kernel(tbl_ref, er_ref, ec_ref, keyr_ref, keyc_ref,
                    sum_ref, cnt_ref, maxp_ref, minn_ref):
    g = pl.program_id(0)
    bi = tbl_ref[2 * g]
    bj = tbl_ref[2 * g + 1]
    B = maxp_ref.shape[0]
    blk = er_ref.shape[0]

    @pl.when(g == 0)
    def _init():
        maxp_ref[...] = jnp.full((B, 1), -1.0, jnp.float32)
        minn_ref[...] = jnp.full((B, 1), _BIG, jnp.float32)

    a = er_ref[...]
    e = ec_ref[...]
    sq_a = jnp.sum(a * a, axis=1, keepdims=True)
    sq_e = jnp.sum(e * e, axis=1, keepdims=True)
    ones = jnp.ones((blk, 1), jnp.float32)
    a_aug = jnp.concatenate([a, sq_a, ones], axis=1)
    e_aug = jnp.concatenate([-2.0 * e, ones, sq_e], axis=1)
    d2 = jax.lax.dot_general(
        a_aug, e_aug, (((1,), (1,)), ((), ())),
        preferred_element_type=jnp.float32)

    key_r = keyr_ref[...]
    key_c = keyc_ref[0, :]
    sbj_r = key_r // 8
    sbj_c = key_c // 8
    key_eq = key_r == key_c[None, :]
    sbj_eq = sbj_r == sbj_c[None, :]
    diff = (jax.lax.broadcasted_iota(jnp.int32, (blk, blk), 1)
            - jax.lax.broadcasted_iota(jnp.int32, (blk, blk), 0))
    ne = diff != (bi - bj) * blk
    pos = key_eq & ne
    neg = sbj_eq & jnp.logical_not(key_eq)

    posval = jnp.where(pos, d2, -1.0)
    negval = jnp.where(neg, d2, _BIG)

    r0 = bi * blk
    c0 = bj * blk
    maxp_ref[pl.ds(r0, blk), :] = jnp.maximum(
        maxp_ref[pl.ds(r0, blk), :],
        jnp.max(posval, axis=1, keepdims=True))
    minn_ref[pl.ds(r0, blk), :] = jnp.minimum(
        minn_ref[pl.ds(r0, blk), :],
        jnp.min(negval, axis=1, keepdims=True))

    @pl.when(bj != bi)
    def _cols():
        cmaxp = jnp.max(posval, axis=0, keepdims=True)
        cminn = jnp.min(negval, axis=0, keepdims=True)
        maxp_ref[pl.ds(c0, blk), :] = jnp.maximum(
            maxp_ref[pl.ds(c0, blk), :], cmaxp.reshape(blk, 1))
        minn_ref[pl.ds(c0, blk), :] = jnp.minimum(
            minn_ref[pl.ds(c0, blk), :], cminn.reshape(blk, 1))

    @pl.when(g == pl.num_programs(0) - 1)
    def _final():
        maxp = maxp_ref[...]
        minn = minn_ref[...]
        valid = (maxp >= 0.0) & (minn < 1e29)
        dp = jnp.sqrt(jnp.maximum(maxp, 0.0))
        dn = jnp.sqrt(jnp.maximum(minn, 0.0))
        per = jnp.maximum(dp - dn + _MARGIN, 0.0)
        sum_ref[...] = jnp.sum(jnp.where(valid, per, 0.0)).reshape(1, 1)
        cnt_ref[...] = jnp.sum(valid.astype(jnp.float32)).reshape(1, 1)


def kernel(emb, labels, sbj):
    B, D = emb.shape
    lbl32 = labels.astype(jnp.int32)
    sbj32 = sbj.astype(jnp.int32)
    key = (sbj32 * 8 + lbl32)
    tbl = jnp.asarray(np.array(_PAIRS, dtype=np.int32).reshape(-1))

    grid_spec = pltpu.PrefetchScalarGridSpec(
        num_scalar_prefetch=1,
        grid=(_NPAIR,),
        in_specs=[
            pl.BlockSpec((_BLK, D), lambda g, t: (t[2 * g], 0)),
            pl.BlockSpec((_BLK, D), lambda g, t: (t[2 * g + 1], 0)),
            pl.BlockSpec((_BLK, 1), lambda g, t: (t[2 * g], 0)),
            pl.BlockSpec((1, _BLK), lambda g, t: (0, t[2 * g + 1])),
        ],
        out_specs=[
            pl.BlockSpec((1, 1), lambda g, t: (0, 0)),
            pl.BlockSpec((1, 1), lambda g, t: (0, 0)),
        ],
        scratch_shapes=[
            pltpu.VMEM((B, 1), jnp.float32),
            pltpu.VMEM((B, 1), jnp.float32),
        ],
    )
    s, c = pl.pallas_call(
        _triplet_kernel,
        grid_spec=grid_spec,
        out_shape=[
            jax.ShapeDtypeStruct((1, 1), jnp.float32),
            jax.ShapeDtypeStruct((1, 1), jnp.float32),
        ],
    )(tbl, emb, emb, key.reshape(B, 1), key.reshape(1, B))
    return s[0, 0] / jnp.maximum(c[0, 0], 1.0)


# speedup vs baseline: 1.9435x; 1.2738x over previous
import jax
import jax.numpy as jnp
from jax.experimental import pallas as pl
from jax.experimental.pallas import tpu as pltpu

_MARGIN = 0.8
_BLK = 512
_BIG = 1e30


def _triplet_kernel(a_ref, e_ref, key_ref, sbj_ref, sum_ref, cnt_ref):
    i = pl.program_id(0)
    a = a_ref[...]
    e = e_ref[...]
    B = e.shape[0]
    blk = a.shape[0]

    sq_blk = jnp.sum(a * a, axis=1, keepdims=True)
    sq_all = jnp.sum(e * e, axis=1, keepdims=True)
    a_aug = jnp.concatenate(
        [a, sq_blk, jnp.ones((blk, 1), jnp.float32)], axis=1)
    e_aug = jnp.concatenate(
        [-2.0 * e, jnp.ones((B, 1), jnp.float32), sq_all], axis=1)
    d2 = jax.lax.dot_general(
        a_aug, e_aug, (((1,), (1,)), ((), ())),
        preferred_element_type=jnp.float32)

    key = key_ref[0, :]
    sbj = sbj_ref[0, :]
    key_r = key_ref[0, pl.ds(i * blk, blk)]
    sbj_r = sbj_ref[0, pl.ds(i * blk, blk)]

    key_eq = key_r[:, None] == key[None, :]
    sbj_eq = sbj_r[:, None] == sbj[None, :]
    row = i * blk + jax.lax.broadcasted_iota(jnp.int32, (blk, B), 0)
    col = jax.lax.broadcasted_iota(jnp.int32, (blk, B), 1)
    pos = key_eq & (row != col)
    neg = sbj_eq & jnp.logical_not(key_eq)

    dpos2 = jnp.max(jnp.where(pos, d2, -1.0), axis=1)
    dneg2 = jnp.min(jnp.where(neg, d2, _BIG), axis=1)
    valid = (dpos2 >= 0.0) & (dneg2 < 1e29)

    dp = jnp.sqrt(jnp.maximum(dpos2, 0.0))
    dn = jnp.sqrt(jnp.maximum(dneg2, 0.0))
    per = jnp.maximum(dp - dn + _MARGIN, 0.0)
    psum = jnp.sum(jnp.where(valid, per, 0.0))
    pcnt = jnp.sum(valid.astype(jnp.float32))

    @pl.when(i == 0)
    def _():
        sum_ref[...] = jnp.zeros((1, 1), jnp.float32)
        cnt_ref[...] = jnp.zeros((1, 1), jnp.float32)

    sum_ref[...] += psum.reshape(1, 1)
    cnt_ref[...] += pcnt.reshape(1, 1)


def kernel(emb, labels, sbj):
    B, D = emb.shape
    lbl32 = labels.astype(jnp.int32)
    sbj32 = sbj.astype(jnp.int32)
    key2 = (sbj32 * 8 + lbl32).reshape(1, B)
    sbj2 = sbj32.reshape(1, B)
    grid = B // _BLK
    s, c = pl.pallas_call(
        _triplet_kernel,
        grid=(grid,),
        in_specs=[
            pl.BlockSpec((_BLK, D), lambda i: (i, 0)),
            pl.BlockSpec((B, D), lambda i: (0, 0)),
            pl.BlockSpec((1, B), lambda i: (0, 0)),
            pl.BlockSpec((1, B), lambda i: (0, 0)),
        ],
        out_specs=[
            pl.BlockSpec((1, 1), lambda i: (0, 0)),
            pl.BlockSpec((1, 1), lambda i: (0, 0)),
        ],
        out_shape=[
            jax.ShapeDtypeStruct((1, 1), jnp.float32),
            jax.ShapeDtypeStruct((1, 1), jnp.float32),
        ],
    )(emb, emb, key2, sbj2)
    return s[0, 0] / jnp.maximum(c[0, 0], 1.0)


# speedup vs baseline: 2.1125x; 1.0870x over previous
import jax
import jax.numpy as jnp
from jax.experimental import pallas as pl
from jax.experimental.pallas import tpu as pltpu

_MARGIN = 0.8
_BLK = 1024
_BIG = 1e30


def _triplet_kernel(a_ref, e_ref, key_ref, sbj_ref, sum_ref, cnt_ref):
    i = pl.program_id(0)
    a = a_ref[...]
    e = e_ref[...]
    B = e.shape[0]
    blk = a.shape[0]

    sq_blk = jnp.sum(a * a, axis=1, keepdims=True)
    sq_all = jnp.sum(e * e, axis=1, keepdims=True)
    a_aug = jnp.concatenate(
        [a, sq_blk, jnp.ones((blk, 1), jnp.float32)], axis=1)
    e_aug = jnp.concatenate(
        [-2.0 * e, jnp.ones((B, 1), jnp.float32), sq_all], axis=1)
    d2 = jax.lax.dot_general(
        a_aug, e_aug, (((1,), (1,)), ((), ())),
        preferred_element_type=jnp.float32)

    key = key_ref[0, :]
    sbj = sbj_ref[0, :]
    key_r = key_ref[0, pl.ds(i * blk, blk)]
    sbj_r = sbj_ref[0, pl.ds(i * blk, blk)]

    key_eq = key_r[:, None] == key[None, :]
    sbj_eq = sbj_r[:, None] == sbj[None, :]
    row = i * blk + jax.lax.broadcasted_iota(jnp.int32, (blk, B), 0)
    col = jax.lax.broadcasted_iota(jnp.int32, (blk, B), 1)
    pos = key_eq & (row != col)
    neg = sbj_eq & jnp.logical_not(key_eq)

    dpos2 = jnp.max(jnp.where(pos, d2, -1.0), axis=1)
    dneg2 = jnp.min(jnp.where(neg, d2, _BIG), axis=1)
    valid = (dpos2 >= 0.0) & (dneg2 < 1e29)

    dp = jnp.sqrt(jnp.maximum(dpos2, 0.0))
    dn = jnp.sqrt(jnp.maximum(dneg2, 0.0))
    per = jnp.maximum(dp - dn + _MARGIN, 0.0)
    psum = jnp.sum(jnp.where(valid, per, 0.0))
    pcnt = jnp.sum(valid.astype(jnp.float32))

    @pl.when(i == 0)
    def _():
        sum_ref[...] = jnp.zeros((1, 1), jnp.float32)
        cnt_ref[...] = jnp.zeros((1, 1), jnp.float32)

    sum_ref[...] += psum.reshape(1, 1)
    cnt_ref[...] += pcnt.reshape(1, 1)


def kernel(emb, labels, sbj):
    B, D = emb.shape
    lbl32 = labels.astype(jnp.int32)
    sbj32 = sbj.astype(jnp.int32)
    key2 = (sbj32 * 8 + lbl32).reshape(1, B)
    sbj2 = sbj32.reshape(1, B)
    grid = B // _BLK
    s, c = pl.pallas_call(
        _triplet_kernel,
        grid=(grid,),
        in_specs=[
            pl.BlockSpec((_BLK, D), lambda i: (i, 0)),
            pl.BlockSpec((B, D), lambda i: (0, 0)),
            pl.BlockSpec((1, B), lambda i: (0, 0)),
            pl.BlockSpec((1, B), lambda i: (0, 0)),
        ],
        out_specs=[
            pl.BlockSpec((1, 1), lambda i: (0, 0)),
            pl.BlockSpec((1, 1), lambda i: (0, 0)),
        ],
        out_shape=[
            jax.ShapeDtypeStruct((1, 1), jnp.float32),
            jax.ShapeDtypeStruct((1, 1), jnp.float32),
        ],
    )(emb, emb, key2, sbj2)
    return s[0, 0] / jnp.maximum(c[0, 0], 1.0)


# speedup vs baseline: 2.2496x; 1.0649x over previous
import jax
import jax.numpy as jnp
from jax.experimental import pallas as pl
from jax.experimental.pallas import tpu as pltpu

_MARGIN = 0.8
_BLK = 2048
_BIG = 1e30


def _triplet_kernel(a_ref, e_ref, key_ref, sbj_ref, sum_ref, cnt_ref):
    i = pl.program_id(0)
    a = a_ref[...]
    e = e_ref[...]
    B = e.shape[0]
    blk = a.shape[0]

    sq_blk = jnp.sum(a * a, axis=1, keepdims=True)
    sq_all = jnp.sum(e * e, axis=1, keepdims=True)
    a_aug = jnp.concatenate(
        [a, sq_blk, jnp.ones((blk, 1), jnp.float32)], axis=1)
    e_aug = jnp.concatenate(
        [-2.0 * e, jnp.ones((B, 1), jnp.float32), sq_all], axis=1)
    d2 = jax.lax.dot_general(
        a_aug, e_aug, (((1,), (1,)), ((), ())),
        preferred_element_type=jnp.float32)

    key = key_ref[0, :]
    sbj = sbj_ref[0, :]
    key_r = key_ref[0, pl.ds(i * blk, blk)]
    sbj_r = sbj_ref[0, pl.ds(i * blk, blk)]

    key_eq = key_r[:, None] == key[None, :]
    sbj_eq = sbj_r[:, None] == sbj[None, :]
    row = i * blk + jax.lax.broadcasted_iota(jnp.int32, (blk, B), 0)
    col = jax.lax.broadcasted_iota(jnp.int32, (blk, B), 1)
    pos = key_eq & (row != col)
    neg = sbj_eq & jnp.logical_not(key_eq)

    dpos2 = jnp.max(jnp.where(pos, d2, -1.0), axis=1)
    dneg2 = jnp.min(jnp.where(neg, d2, _BIG), axis=1)
    valid = (dpos2 >= 0.0) & (dneg2 < 1e29)

    dp = jnp.sqrt(jnp.maximum(dpos2, 0.0))
    dn = jnp.sqrt(jnp.maximum(dneg2, 0.0))
    per = jnp.maximum(dp - dn + _MARGIN, 0.0)
    psum = jnp.sum(jnp.where(valid, per, 0.0))
    pcnt = jnp.sum(valid.astype(jnp.float32))

    @pl.when(i == 0)
    def _():
        sum_ref[...] = jnp.zeros((1, 1), jnp.float32)
        cnt_ref[...] = jnp.zeros((1, 1), jnp.float32)

    sum_ref[...] += psum.reshape(1, 1)
    cnt_ref[...] += pcnt.reshape(1, 1)


def kernel(emb, labels, sbj):
    B, D = emb.shape
    lbl32 = labels.astype(jnp.int32)
    sbj32 = sbj.astype(jnp.int32)
    key2 = (sbj32 * 8 + lbl32).reshape(1, B)
    sbj2 = sbj32.reshape(1, B)
    grid = B // _BLK
    s, c = pl.pallas_call(
        _triplet_kernel,
        grid=(grid,),
        in_specs=[
            pl.BlockSpec((_BLK, D), lambda i: (i, 0)),
            pl.BlockSpec((B, D), lambda i: (0, 0)),
            pl.BlockSpec((1, B), lambda i: (0, 0)),
            pl.BlockSpec((1, B), lambda i: (0, 0)),
        ],
        out_specs=[
            pl.BlockSpec((1, 1), lambda i: (0, 0)),
            pl.BlockSpec((1, 1), lambda i: (0, 0)),
        ],
        out_shape=[
            jax.ShapeDtypeStruct((1, 1), jnp.float32),
            jax.ShapeDtypeStruct((1, 1), jnp.float32),
        ],
    )(emb, emb, key2, sbj2)
    return s[0, 0] / jnp.maximum(c[0, 0], 1.0)
